# baseline (speedup 1.0000x reference)
import functools

import jax
import jax.numpy as jnp
from jax import lax
from jax.experimental import pallas as pl
from jax.experimental.pallas import tpu as pltpu
from jax.experimental.pallas import tpu_sc as plsc

N = 10000
NPAD = 10240
D = 128
K = 8
DK = 16
F = K * DK
T = 4
E = 80000
NC = 2
NS = 16
ROWS_PT = NPAD // NS
EDGES_PT = E // NS
CH = 128
NCH = 40
EDGES_PT_PAD = NCH * CH
NBUF = 2
RB = 1000
EPS = 1e-12


def _dense_init_body(x_ref, w_ref, b_ref, m_ref, o_ref):
    y = jnp.dot(x_ref[...], w_ref[...], preferred_element_type=jnp.float32)
    y = jnp.tanh(y + b_ref[...])
    s = jnp.dot(y * y, m_ref[...], preferred_element_type=jnp.float32)
    o_ref[...] = y / jnp.maximum(jnp.sqrt(s), EPS)


def _dense_combine_body(f_ref, a_ref, w_ref, b_ref, m_ref, o_ref):
    acc = jnp.dot(f_ref[...], w_ref[0], preferred_element_type=jnp.float32)
    for t in range(T):
        acc = acc + jnp.dot(a_ref[t], w_ref[t + 1],
                            preferred_element_type=jnp.float32)
    y = jnp.tanh(acc + b_ref[...])
    s = jnp.dot(y * y, m_ref[...], preferred_element_type=jnp.float32)
    o_ref[...] = y / jnp.maximum(jnp.sqrt(s), EPS)


def _dense_init(x, w, b, m):
    return pl.pallas_call(
        _dense_init_body,
        grid=(N // RB,),
        in_specs=[
            pl.BlockSpec((RB, D), lambda i: (i, 0)),
            pl.BlockSpec((D, F), lambda i: (0, 0)),
            pl.BlockSpec((1, F), lambda i: (0, 0)),
            pl.BlockSpec((F, F), lambda i: (0, 0)),
        ],
        out_specs=pl.BlockSpec((RB, F), lambda i: (i, 0)),
        out_shape=jax.ShapeDtypeStruct((N, F), jnp.float32),
    )(x, w, b, m)


def _dense_combine(f, aggs, wbd, b, m):
    return pl.pallas_call(
        _dense_combine_body,
        grid=(N // RB,),
        in_specs=[
            pl.BlockSpec((RB, F), lambda i: (i, 0)),
            pl.BlockSpec((T, RB, F), lambda i: (0, i, 0)),
            pl.BlockSpec((T + 1, F, F), lambda i: (0, 0, 0)),
            pl.BlockSpec((1, F), lambda i: (0, 0)),
            pl.BlockSpec((F, F), lambda i: (0, 0)),
        ],
        out_specs=pl.BlockSpec((RB, F), lambda i: (i, 0)),
        out_shape=jax.ShapeDtypeStruct((N, F), jnp.float32),
    )(f, aggs, wbd, b, m)


def _sc_agg_body(f_hbm, edges_hbm, zeros_hbm, out_hbm,
                 src_v, dst_v, r0_v, r1_v, agg_sh, sem0, sem1):
    c = lax.axis_index("c")
    s = lax.axis_index("s")
    row0 = s * ROWS_PT
    rows = (r0_v, r1_v)
    sems = (sem0, sem1)

    def gissue(j, b):
        pltpu.async_copy(f_hbm.at[src_v.at[j]], rows[b], sems[b])

    def gwait(j, b):
        pltpu.make_async_copy(f_hbm.at[src_v.at[j]], rows[b], sems[b]).wait()

    for rnd in range(T // NC):
        t = c * (T // NC) + rnd
        pltpu.sync_copy(zeros_hbm, agg_sh.at[pl.ds(row0, ROWS_PT)])
        pltpu.sync_copy(edges_hbm.at[t, 0, s], src_v)
        pltpu.sync_copy(edges_hbm.at[t, 1, s], dst_v)
        plsc.subcore_barrier()

        for b in range(NBUF):
            gissue(b, b)

        def step(i, carry):
            for b in range(NBUF):
                j = i * NBUF + b
                gwait(j, b)
                pltpu.sync_copy(rows[b], agg_sh.at[dst_v.at[j]], add=True)

                @pl.when(j + NBUF < NCH)
                def _(j=j, b=b):
                    gissue(j + NBUF, b)
            return carry

        lax.fori_loop(0, NCH // NBUF, step, 0)
        plsc.subcore_barrier()
        pltpu.sync_copy(agg_sh.at[pl.ds(row0, ROWS_PT)],
                        out_hbm.at[t, pl.ds(row0, ROWS_PT)])


@functools.cache
def _sc_agg_kernel():
    return pl.kernel(
        _sc_agg_body,
        out_type=jax.ShapeDtypeStruct((T, NPAD, F), jnp.float32),
        mesh=plsc.VectorSubcoreMesh(
            core_axis_name="c", subcore_axis_name="s",
            num_cores=NC, num_subcores=NS),
        scratch_types=[
            pltpu.VMEM((NCH, CH), jnp.int32),
            pltpu.VMEM((NCH, CH), jnp.int32),
        ] + [pltpu.VMEM((CH, F), jnp.float32)] * NBUF + [
            pltpu.VMEM_SHARED((NPAD, F), jnp.float32),
        ] + [pltpu.SemaphoreType.DMA] * NBUF,
    )


def _pad_edges(edges_each_type):
    e = edges_each_type.astype(jnp.int32).reshape(T, 2, NS, EDGES_PT)
    pad = EDGES_PT_PAD - EDGES_PT
    src = jnp.pad(e[:, 0], ((0, 0), (0, 0), (0, pad)))
    dst = jnp.pad(e[:, 1], ((0, 0), (0, 0), (0, pad)), constant_values=N)
    return jnp.stack([src, dst], axis=1).reshape(T, 2, NS, NCH, CH)


def _sc_agg(f, edges, zeros):
    return _sc_agg_kernel()(f, edges, zeros)


def _block_diag_weights(w):
    w5 = w.reshape(K, T + 1, DK, DK).transpose(1, 0, 2, 3)
    eye = jnp.eye(K, dtype=w.dtype)
    wbd = w5[:, :, :, None, :] * eye[None, :, None, :, None]
    return wbd.reshape(T + 1, F, F)


def kernel(X, edges_each_type, disen_weights, disen_bias,
           conv_W_0, conv_b_0, conv_W_1, conv_b_1):
    edges = _pad_edges(edges_each_type)
    zeros = jnp.zeros((ROWS_PT, F), jnp.float32)
    lanes = jnp.arange(F) % DK
    m = (lanes[:, None] == lanes[None, :]).astype(jnp.float32)

    wd = disen_weights.transpose(1, 0, 2).reshape(D, F)
    f = _dense_init(X, wd, disen_bias.reshape(1, F), m)

    for w, b in ((conv_W_0, conv_b_0), (conv_W_1, conv_b_1)):
        aggs = _sc_agg(f, edges, zeros)
        f = _dense_combine(f, aggs, _block_diag_weights(w), b.reshape(1, F), m)

    return f.reshape(N, K, DK)


# speedup vs baseline: 2.2404x; 2.2404x over previous
import functools

import jax
import jax.numpy as jnp
from jax import lax
from jax.experimental import pallas as pl
from jax.experimental.pallas import tpu as pltpu
from jax.experimental.pallas import tpu_sc as plsc

N = 10000
NPAD = 10240
D = 128
K = 8
DK = 16
F = K * DK
T = 4
E = 80000
NC = 2
NS = 16
ROWS_PT = NPAD // NS
EDGES_PT = E // NS
CH = 50
NCH = 100
EDGES_PT_PAD = NCH * CH
NBUF = 2
RB = 1000
EPS = 1e-12


def _dense_init_body(x_ref, w_ref, b_ref, m_ref, o_ref):
    y = jnp.dot(x_ref[...], w_ref[...], preferred_element_type=jnp.float32)
    y = jnp.tanh(y + b_ref[...])
    s = jnp.dot(y * y, m_ref[...], preferred_element_type=jnp.float32)
    o_ref[...] = y / jnp.maximum(jnp.sqrt(s), EPS)


def _dense_combine_body(f_ref, a_ref, w_ref, b_ref, m_ref, o_ref):
    acc = jnp.dot(f_ref[...], w_ref[0], preferred_element_type=jnp.float32)
    for t in range(T):
        acc = acc + jnp.dot(a_ref[t], w_ref[t + 1],
                            preferred_element_type=jnp.float32)
    y = jnp.tanh(acc + b_ref[...])
    s = jnp.dot(y * y, m_ref[...], preferred_element_type=jnp.float32)
    o_ref[...] = y / jnp.maximum(jnp.sqrt(s), EPS)


def _dense_init(x, w, b, m):
    return pl.pallas_call(
        _dense_init_body,
        grid=(N // RB,),
        in_specs=[
            pl.BlockSpec((RB, D), lambda i: (i, 0)),
            pl.BlockSpec((D, F), lambda i: (0, 0)),
            pl.BlockSpec((1, F), lambda i: (0, 0)),
            pl.BlockSpec((F, F), lambda i: (0, 0)),
        ],
        out_specs=pl.BlockSpec((RB, F), lambda i: (i, 0)),
        out_shape=jax.ShapeDtypeStruct((N, F), jnp.float32),
    )(x, w, b, m)


def _dense_combine(f, aggs, wbd, b, m):
    return pl.pallas_call(
        _dense_combine_body,
        grid=(N // RB,),
        in_specs=[
            pl.BlockSpec((RB, F), lambda i: (i, 0)),
            pl.BlockSpec((T, RB, F), lambda i: (0, i, 0)),
            pl.BlockSpec((T + 1, F, F), lambda i: (0, 0, 0)),
            pl.BlockSpec((1, F), lambda i: (0, 0)),
            pl.BlockSpec((F, F), lambda i: (0, 0)),
        ],
        out_specs=pl.BlockSpec((RB, F), lambda i: (i, 0)),
        out_shape=jax.ShapeDtypeStruct((N, F), jnp.float32),
    )(f, aggs, wbd, b, m)


def _sc_agg_body(f_hbm, edges_hbm, zeros_hbm, out_hbm,
                 src_v, dst_v, r0_v, r1_v, agg_sh, sem0, sem1):
    c = lax.axis_index("c")
    s = lax.axis_index("s")
    row0 = s * ROWS_PT
    rows = (r0_v, r1_v)
    sems = (sem0, sem1)

    def gissue(j, b):
        pltpu.async_copy(f_hbm.at[src_v.at[j]], rows[b], sems[b])

    def gwait(j, b):
        pltpu.make_async_copy(f_hbm.at[src_v.at[j]], rows[b], sems[b]).wait()

    for rnd in range(T // NC):
        t = c * (T // NC) + rnd
        pltpu.sync_copy(zeros_hbm, agg_sh.at[pl.ds(row0, ROWS_PT)])
        pltpu.sync_copy(edges_hbm.at[t, 0, s], src_v)
        pltpu.sync_copy(edges_hbm.at[t, 1, s], dst_v)
        plsc.subcore_barrier()

        for b in range(NBUF):
            gissue(b, b)

        def step(i, carry):
            for b in range(NBUF):
                j = i * NBUF + b
                gwait(j, b)
                pltpu.sync_copy(rows[b], agg_sh.at[dst_v.at[j]], add=True)

                @pl.when(j + NBUF < NCH)
                def _(j=j, b=b):
                    gissue(j + NBUF, b)
            return carry

        lax.fori_loop(0, NCH // NBUF, step, 0)
        plsc.subcore_barrier()
        pltpu.sync_copy(agg_sh.at[pl.ds(row0, ROWS_PT)],
                        out_hbm.at[t, pl.ds(row0, ROWS_PT)])


@functools.cache
def _sc_agg_kernel():
    return pl.kernel(
        _sc_agg_body,
        out_type=jax.ShapeDtypeStruct((T, NPAD, F), jnp.float32),
        mesh=plsc.VectorSubcoreMesh(
            core_axis_name="c", subcore_axis_name="s",
            num_cores=NC, num_subcores=NS),
        scratch_types=[
            pltpu.VMEM((NCH, CH), jnp.int32),
            pltpu.VMEM((NCH, CH), jnp.int32),
        ] + [pltpu.VMEM((CH, F), jnp.float32)] * NBUF + [
            pltpu.VMEM_SHARED((NPAD, F), jnp.float32),
        ] + [pltpu.SemaphoreType.DMA] * NBUF,
    )


def _pad_edges(edges_each_type):
    e = edges_each_type.astype(jnp.int32).reshape(T, 2, NS, EDGES_PT)
    pad = EDGES_PT_PAD - EDGES_PT
    src = jnp.pad(e[:, 0], ((0, 0), (0, 0), (0, pad)))
    dst = jnp.pad(e[:, 1], ((0, 0), (0, 0), (0, pad)), constant_values=N)
    return jnp.stack([src, dst], axis=1).reshape(T, 2, NS, NCH, CH)


def _sc_agg(f, edges, zeros):
    return _sc_agg_kernel()(f, edges, zeros)


def _block_diag_weights(w):
    w5 = w.reshape(K, T + 1, DK, DK).transpose(1, 0, 2, 3)
    eye = jnp.eye(K, dtype=w.dtype)
    wbd = w5[:, :, :, None, :] * eye[None, :, None, :, None]
    return wbd.reshape(T + 1, F, F)


def kernel(X, edges_each_type, disen_weights, disen_bias,
           conv_W_0, conv_b_0, conv_W_1, conv_b_1):
    edges = _pad_edges(edges_each_type)
    zeros = jnp.zeros((ROWS_PT, F), jnp.float32)
    lanes = jnp.arange(F) % DK
    m = (lanes[:, None] == lanes[None, :]).astype(jnp.float32)

    wd = disen_weights.transpose(1, 0, 2).reshape(D, F)
    f = _dense_init(X, wd, disen_bias.reshape(1, F), m)

    for w, b in ((conv_W_0, conv_b_0), (conv_W_1, conv_b_1)):
        aggs = _sc_agg(f, edges, zeros)
        f = _dense_combine(f, aggs, _block_diag_weights(w), b.reshape(1, F), m)

    return f.reshape(N, K, DK)
